# baseline (speedup 1.0000x reference)
import functools
import math

import jax
import jax.numpy as jnp
from jax import lax
from jax.experimental import pallas as pl
from jax.experimental.pallas import tpu as pltpu
from jax.experimental.pallas import tpu_sc as plsc

_LANES = 16
_CHUNK = 128
_NBUF = 2


def _build_lookup(total, n_chunks, d_model, vocab):
    info = plsc.get_sparse_core_info()
    nc, ns = info.num_cores, info.num_subcores
    nw = nc * ns
    per_w = total // nw
    scale = math.sqrt(d_model)
    n_groups = n_chunks // _NBUF

    mesh = plsc.VectorSubcoreMesh(core_axis_name="c", subcore_axis_name="s")

    @functools.partial(
        pl.kernel,
        mesh=mesh,
        compiler_params=pltpu.CompilerParams(use_tc_tiling_on_sc=False),
        out_type=jax.ShapeDtypeStruct((total, d_model), jnp.float32),
        scratch_types=[
            pltpu.VMEM((n_chunks, _CHUNK), jnp.int32),
            pltpu.VMEM((_NBUF, _CHUNK, d_model), jnp.float32),
            pltpu.VMEM((_NBUF, _CHUNK, d_model), jnp.float32),
            pltpu.SemaphoreType.DMA,
            pltpu.SemaphoreType.DMA,
            pltpu.SemaphoreType.DMA,
            pltpu.SemaphoreType.DMA,
        ],
    )
    def run(x_hbm, lut_hbm, out_hbm, idx_v, gbuf, obuf, gs0, gs1, os0, os1):
        gsems = [gs0, gs1]
        osems = [os0, os1]
        wid = lax.axis_index("s") * nc + lax.axis_index("c")
        base = wid * per_w
        pltpu.sync_copy(x_hbm.at[wid], idx_v)

        def gather(c, b):
            pltpu.async_copy(lut_hbm.at[idx_v.at[c]], gbuf.at[b], gsems[b])

        def gwait(c, b):
            pltpu.make_async_copy(
                lut_hbm.at[idx_v.at[c]], gbuf.at[b], gsems[b]
            ).wait()

        def out_slice(c):
            return out_hbm.at[pl.ds(base + c * _CHUNK, _CHUNK)]

        def scatter(c, b):
            pltpu.async_copy(obuf.at[b], out_slice(c), osems[b])

        def owait(c, b):
            pltpu.make_async_copy(obuf.at[b], out_slice(c), osems[b]).wait()

        def scale_chunk(b):
            @pl.loop(0, _CHUNK, unroll=4)
            def row_body(r):
                for j in range(d_model // _LANES):
                    sl = pl.ds(j * _LANES, _LANES)
                    obuf[b, r, sl] = gbuf[b, r, sl] * scale

        for b in range(_NBUF):
            gather(b, b)

        for b in range(_NBUF):
            gwait(b, b)
            scale_chunk(b)
            gather(b + _NBUF, b)
            scatter(b, b)

        def group_body(g, carry):
            for b in range(_NBUF):
                c = g * _NBUF + b
                gwait(c, b)
                owait(c - _NBUF, b)
                scale_chunk(b)
                gather(c + _NBUF, b)
                scatter(c, b)
            return carry

        lax.fori_loop(1, n_groups - 1, group_body, 0)

        for b in range(_NBUF):
            c = (n_groups - 1) * _NBUF + b
            gwait(c, b)
            owait(c - _NBUF, b)
            scale_chunk(b)
            scatter(c, b)

        for b in range(_NBUF):
            c = (n_groups - 1) * _NBUF + b
            owait(c, b)

    return run


def kernel(x, lut):
    b, t = x.shape
    vocab, d_model = lut.shape
    total = b * t

    info = plsc.get_sparse_core_info()
    nw = info.num_cores * info.num_subcores
    per_w = total // nw
    n_chunks = per_w // _CHUNK

    x_resh = x.reshape(nw, n_chunks, _CHUNK).astype(jnp.int32)
    run = _build_lookup(total, n_chunks, d_model, vocab)
    out = run(x_resh, lut)
    return out.reshape(b, t, d_model)
